# Initial kernel scaffold; baseline (speedup 1.0000x reference)
import jax
import jax.numpy as jnp
from jax.experimental import pallas as pl


def kernel(x, edge_index, W, b):
    raise NotImplementedError("write your pallas kernel here")


# speedup vs baseline: 7.8333x; 7.8333x over previous
import dataclasses
import functools

import jax
import jax.numpy as jnp
from jax import lax
from jax.experimental import pallas as pl
from jax.experimental.pallas import tpu as pltpu
from jax.experimental.pallas import tpu_sc as plsc

N = 10000
E = 320000
D = 128

NC = 2
NS = 16
NW = NC * NS
EPW = E // NW
BATCH = 128
NFULL = EPW // BATCH
TAIL = EPW - NFULL * BATCH
RPW = 624
REM = N - NS * RPW
ZCH = 208
ECH = 2000

_mesh = lambda: plsc.VectorSubcoreMesh(core_axis_name="c", subcore_axis_name="s")


def _sc_params():
    cp = pltpu.CompilerParams()
    if "needs_layout_passes" in pltpu.CompilerParams.__dataclass_fields__:
        cp = dataclasses.replace(cp, needs_layout_passes=False)
    return cp


def _deg_call(src, dst):
    @functools.partial(
        pl.kernel,
        out_type=(
            jax.ShapeDtypeStruct((NC, NS, N), jnp.float32),
            jax.ShapeDtypeStruct((NC, NS, N), jnp.float32),
        ),
        mesh=_mesh(),
        compiler_params=_sc_params(),
        scratch_types=[
            pltpu.VMEM((ECH,), jnp.int32),
            pltpu.VMEM((ECH,), jnp.int32),
            pltpu.VMEM((N,), jnp.float32),
            pltpu.VMEM((N,), jnp.float32),
        ],
    )
    def deg_kernel(src_hbm, dst_hbm, outs_hbm, outd_hbm, sbuf, dbuf,
                   shist, dhist):
        core = lax.axis_index("c")
        sid = lax.axis_index("s")
        wid = sid * NC + core

        @pl.loop(0, N, step=16)
        def _(r):
            shist[pl.ds(r, 16)] = jnp.zeros((16,), jnp.float32)
            dhist[pl.ds(r, 16)] = jnp.zeros((16,), jnp.float32)

        ebase = wid * EPW
        ones16 = jnp.ones((16,), jnp.float32)

        @pl.loop(0, EPW // ECH)
        def _(c):
            off = ebase + c * ECH
            pltpu.sync_copy(src_hbm.at[pl.ds(off, ECH)], sbuf)
            pltpu.sync_copy(dst_hbm.at[pl.ds(off, ECH)], dbuf)

            @pl.loop(0, ECH, step=16)
            def _(k):
                plsc.addupdate_scatter(shist, [sbuf[pl.ds(k, 16)]], ones16)
                plsc.addupdate_scatter(dhist, [dbuf[pl.ds(k, 16)]], ones16)

        pltpu.sync_copy(shist, outs_hbm.at[core].at[sid])
        pltpu.sync_copy(dhist, outd_hbm.at[core].at[sid])

    return deg_kernel(src, dst)


def _agg_call(h, src, dst):
    @functools.partial(
        pl.kernel,
        out_type=jax.ShapeDtypeStruct((NC, N, D), jnp.float32),
        mesh=_mesh(),
        scratch_types=[
            pltpu.VMEM((BATCH,), jnp.int32),
            pltpu.VMEM((BATCH,), jnp.int32),
            pltpu.VMEM((TAIL,), jnp.int32),
            pltpu.VMEM((TAIL,), jnp.int32),
            pltpu.VMEM((BATCH, D), jnp.float32),
            pltpu.VMEM((TAIL, D), jnp.float32),
            pltpu.VMEM((ZCH, D), jnp.float32),
            pltpu.VMEM_SHARED((N, D), jnp.float32),
            pltpu.SemaphoreType.DMA,
        ],
    )
    def agg_kernel(h_hbm, src_hbm, dst_hbm, out_hbm, sidx, didx, sidx_t,
                   didx_t, rows_v, rows_t, zero_v, agg_sh, sem):
        core = lax.axis_index("c")
        sid = lax.axis_index("s")
        wid = sid * NC + core

        @pl.loop(0, ZCH)
        def _(r):
            @pl.loop(0, D, step=16)
            def _(c):
                zero_v[r, pl.ds(c, 16)] = jnp.zeros((16,), jnp.float32)

        base_row = sid * RPW

        @pl.loop(0, 3)
        def _(k):
            pltpu.sync_copy(zero_v, agg_sh.at[pl.ds(base_row + k * ZCH, ZCH)])

        @pl.when(sid == NS - 1)
        def _():
            pltpu.sync_copy(zero_v.at[pl.ds(0, REM)],
                            agg_sh.at[pl.ds(NS * RPW, REM)])

        plsc.subcore_barrier()

        ebase = wid * EPW

        @pl.loop(0, NFULL)
        def _(j):
            off = ebase + j * BATCH
            pltpu.sync_copy(src_hbm.at[pl.ds(off, BATCH)], sidx)
            pltpu.async_copy(h_hbm.at[sidx], rows_v, sem).wait()
            pltpu.sync_copy(dst_hbm.at[pl.ds(off, BATCH)], didx)
            pltpu.sync_copy(rows_v, agg_sh.at[didx], add=True)

        toff = ebase + NFULL * BATCH
        pltpu.sync_copy(src_hbm.at[pl.ds(toff, TAIL)], sidx_t)
        pltpu.async_copy(h_hbm.at[sidx_t], rows_t, sem).wait()
        pltpu.sync_copy(dst_hbm.at[pl.ds(toff, TAIL)], didx_t)
        pltpu.sync_copy(rows_t, agg_sh.at[didx_t], add=True)

        plsc.subcore_barrier()
        pltpu.sync_copy(agg_sh.at[pl.ds(base_row, RPW)],
                        out_hbm.at[core].at[pl.ds(base_row, RPW)])

        @pl.when(sid == NS - 1)
        def _():
            pltpu.sync_copy(agg_sh.at[pl.ds(NS * RPW, REM)],
                            out_hbm.at[core].at[pl.ds(NS * RPW, REM)])

    return agg_kernel(h, src, dst)


_BN = 2000


def _h_body(x_ref, degp_ref, h_ref):
    deg = jnp.sum(degp_ref[...], axis=1)
    s = lax.rsqrt(jnp.maximum(deg, 1.0))
    h_ref[...] = x_ref[...] * s[:, None]


def _h_call(x, degp_t):
    return pl.pallas_call(
        _h_body,
        grid=(N // _BN,),
        in_specs=[
            pl.BlockSpec((_BN, D), lambda i: (i, 0)),
            pl.BlockSpec((_BN, NW), lambda i: (i, 0)),
        ],
        out_specs=pl.BlockSpec((_BN, D), lambda i: (i, 0)),
        out_shape=jax.ShapeDtypeStruct((N, D), jnp.float32),
    )(x, degp_t)


def _final_body(agg_ref, degp_ref, x_ref, w_ref, b_ref, o_ref):
    agg = agg_ref[0] + agg_ref[1]
    deg = jnp.sum(degp_ref[...], axis=1)
    ndst = lax.rsqrt(jnp.maximum(deg, 1.0))
    a = agg * ndst[:, None]
    x1 = jnp.dot(a, w_ref[...], preferred_element_type=jnp.float32)
    x1 = x1 + b_ref[0][None, :]
    mean = jnp.mean(x1, axis=1, keepdims=True)
    var = jnp.mean((x1 - mean) * (x1 - mean), axis=1, keepdims=True)
    x1 = (x1 - mean) * lax.rsqrt(var + 1e-5)
    o_ref[...] = jnp.maximum(x1, 0.0) + x_ref[...]


def _final_call(aggparts, degp_t, x, W, b):
    return pl.pallas_call(
        _final_body,
        grid=(N // _BN,),
        in_specs=[
            pl.BlockSpec((NC, _BN, D), lambda i: (0, i, 0)),
            pl.BlockSpec((_BN, NW), lambda i: (i, 0)),
            pl.BlockSpec((_BN, D), lambda i: (i, 0)),
            pl.BlockSpec((D, D), lambda i: (0, 0)),
            pl.BlockSpec((1, D), lambda i: (0, 0)),
        ],
        out_specs=pl.BlockSpec((_BN, D), lambda i: (i, 0)),
        out_shape=jax.ShapeDtypeStruct((N, D), jnp.float32),
    )(aggparts, degp_t, x, W, b.reshape(1, D))


def kernel(x, edge_index, W, b):
    src = edge_index[0]
    dst = edge_index[1]
    degs, degd = _deg_call(src, dst)
    degs_t = degs.reshape(NW, N).T
    degd_t = degd.reshape(NW, N).T
    h = _h_call(x, degs_t)
    aggparts = _agg_call(h, src, dst)
    return _final_call(aggparts, degd_t, x, W, b)


# speedup vs baseline: 9.0929x; 1.1608x over previous
import dataclasses
import functools

import jax
import jax.numpy as jnp
from jax import lax
from jax.experimental import pallas as pl
from jax.experimental.pallas import tpu as pltpu
from jax.experimental.pallas import tpu_sc as plsc

N = 10000
E = 320000
D = 128

NC = 2
NS = 16
NW = NC * NS
EPW = E // NW
BATCH = 128
NFULL = EPW // BATCH
TAIL = EPW - NFULL * BATCH
RPW = 624
REM = N - NS * RPW
ZCH = 78
ECH = 2000

_mesh = lambda: plsc.VectorSubcoreMesh(core_axis_name="c", subcore_axis_name="s")


def _sc_params():
    cp = pltpu.CompilerParams()
    if "needs_layout_passes" in pltpu.CompilerParams.__dataclass_fields__:
        cp = dataclasses.replace(cp, needs_layout_passes=False)
    return cp


def _deg_call(src, dst):
    @functools.partial(
        pl.kernel,
        out_type=(
            jax.ShapeDtypeStruct((NC, NS, N), jnp.float32),
            jax.ShapeDtypeStruct((NC, NS, N), jnp.float32),
        ),
        mesh=_mesh(),
        compiler_params=_sc_params(),
        scratch_types=[
            pltpu.VMEM((ECH,), jnp.int32),
            pltpu.VMEM((ECH,), jnp.int32),
            pltpu.VMEM((N,), jnp.float32),
            pltpu.VMEM((N,), jnp.float32),
        ],
    )
    def deg_kernel(src_hbm, dst_hbm, outs_hbm, outd_hbm, sbuf, dbuf,
                   shist, dhist):
        core = lax.axis_index("c")
        sid = lax.axis_index("s")
        wid = sid * NC + core

        @pl.loop(0, N, step=16)
        def _(r):
            shist[pl.ds(r, 16)] = jnp.zeros((16,), jnp.float32)
            dhist[pl.ds(r, 16)] = jnp.zeros((16,), jnp.float32)

        ebase = wid * EPW
        ones16 = jnp.ones((16,), jnp.float32)

        @pl.loop(0, EPW // ECH)
        def _(c):
            off = ebase + c * ECH
            pltpu.sync_copy(src_hbm.at[pl.ds(off, ECH)], sbuf)
            pltpu.sync_copy(dst_hbm.at[pl.ds(off, ECH)], dbuf)

            @pl.loop(0, ECH, step=16)
            def _(k):
                plsc.addupdate_scatter(shist, [sbuf[pl.ds(k, 16)]], ones16)
                plsc.addupdate_scatter(dhist, [dbuf[pl.ds(k, 16)]], ones16)

        pltpu.sync_copy(shist, outs_hbm.at[core].at[sid])
        pltpu.sync_copy(dhist, outd_hbm.at[core].at[sid])

    return deg_kernel(src, dst)


NBW = 80
EPAD = NW * NBW * BATCH
NPAD = 8


def _agg_call(hp, srcp, dst2p):
    @functools.partial(
        pl.kernel,
        out_type=jax.ShapeDtypeStruct((NC, N, D), jnp.float32),
        mesh=_mesh(),
        scratch_types=[
            pltpu.VMEM((NBW * BATCH,), jnp.int32),
            pltpu.VMEM((NBW, BATCH), jnp.int32),
            pltpu.VMEM((BATCH, D), jnp.float32),
            pltpu.VMEM((ZCH, D), jnp.float32),
            pltpu.VMEM_SHARED((N, D), jnp.float32),
            pltpu.SemaphoreType.DMA,
        ],
    )
    def agg_kernel(h_hbm, src_hbm, dst2_hbm, out_hbm, sbuf, dbuf, rows_a,
                   zero_v, agg_sh, sem_a):
        core = lax.axis_index("c")
        sid = lax.axis_index("s")
        wid = sid * NC + core

        def gather_start(j, rows, sem):
            pltpu.async_copy(h_hbm.at[sbuf.at[pl.ds(j * BATCH, BATCH)]],
                             rows, sem)

        def gather_wait(rows, sem):
            pltpu.make_async_copy(
                h_hbm.at[sbuf.at[pl.ds(0, BATCH)]], rows, sem).wait()

        def scatter(j, rows):
            pltpu.sync_copy(rows, agg_sh.at[dbuf.at[j]], add=True)

        pltpu.sync_copy(src_hbm.at[pl.ds(wid * NBW * BATCH, NBW * BATCH)],
                        sbuf)
        pltpu.sync_copy(dst2_hbm.at[pl.ds(wid * NBW, NBW)], dbuf)

        @pl.loop(0, ZCH)
        def _(r):
            @pl.loop(0, D, step=16)
            def _(c):
                zero_v[r, pl.ds(c, 16)] = jnp.zeros((16,), jnp.float32)

        base_row = sid * RPW

        @pl.loop(0, 8)
        def _(k):
            pltpu.sync_copy(zero_v, agg_sh.at[pl.ds(base_row + k * ZCH, ZCH)])

        @pl.when(sid == NS - 1)
        def _():
            pltpu.sync_copy(zero_v.at[pl.ds(0, REM)],
                            agg_sh.at[pl.ds(NS * RPW, REM)])

        plsc.subcore_barrier()

        @pl.loop(0, NBW)
        def _(j):
            pltpu.async_copy(h_hbm.at[sbuf.at[pl.ds(j * BATCH, BATCH)]],
                             rows_a, sem_a).wait()
            scatter(j, rows_a)

        plsc.subcore_barrier()
        pltpu.sync_copy(agg_sh.at[pl.ds(base_row, RPW)],
                        out_hbm.at[core].at[pl.ds(base_row, RPW)])

        @pl.when(sid == NS - 1)
        def _():
            pltpu.sync_copy(agg_sh.at[pl.ds(NS * RPW, REM)],
                            out_hbm.at[core].at[pl.ds(NS * RPW, REM)])

    return agg_kernel(hp, srcp, dst2p)


_BN = 2000


def _h_body(x_ref, degp_ref, h_ref):
    deg = jnp.sum(degp_ref[...], axis=1)
    s = lax.rsqrt(jnp.maximum(deg, 1.0))
    h_ref[...] = x_ref[...] * s[:, None]


def _h_call(x, degp_t):
    return pl.pallas_call(
        _h_body,
        grid=(N // _BN,),
        in_specs=[
            pl.BlockSpec((_BN, D), lambda i: (i, 0)),
            pl.BlockSpec((_BN, NW), lambda i: (i, 0)),
        ],
        out_specs=pl.BlockSpec((_BN, D), lambda i: (i, 0)),
        out_shape=jax.ShapeDtypeStruct((N, D), jnp.float32),
    )(x, degp_t)


def _final_body(agg_ref, degp_ref, x_ref, w_ref, b_ref, o_ref):
    agg = agg_ref[0] + agg_ref[1]
    deg = jnp.sum(degp_ref[...], axis=1)
    ndst = lax.rsqrt(jnp.maximum(deg, 1.0))
    a = agg * ndst[:, None]
    x1 = jnp.dot(a, w_ref[...], preferred_element_type=jnp.float32)
    x1 = x1 + b_ref[0][None, :]
    mean = jnp.mean(x1, axis=1, keepdims=True)
    var = jnp.mean((x1 - mean) * (x1 - mean), axis=1, keepdims=True)
    x1 = (x1 - mean) * lax.rsqrt(var + 1e-5)
    o_ref[...] = jnp.maximum(x1, 0.0) + x_ref[...]


def _final_call(aggparts, degp_t, x, W, b):
    return pl.pallas_call(
        _final_body,
        grid=(N // _BN,),
        in_specs=[
            pl.BlockSpec((NC, _BN, D), lambda i: (0, i, 0)),
            pl.BlockSpec((_BN, NW), lambda i: (i, 0)),
            pl.BlockSpec((_BN, D), lambda i: (i, 0)),
            pl.BlockSpec((D, D), lambda i: (0, 0)),
            pl.BlockSpec((1, D), lambda i: (0, 0)),
        ],
        out_specs=pl.BlockSpec((_BN, D), lambda i: (i, 0)),
        out_shape=jax.ShapeDtypeStruct((N, D), jnp.float32),
    )(aggparts, degp_t, x, W, b.reshape(1, D))


def kernel(x, edge_index, W, b):
    src = edge_index[0]
    dst = edge_index[1]
    degs, degd = _deg_call(src, dst)
    degs_t = degs.reshape(NW, N).T
    degd_t = degd.reshape(NW, N).T
    h = _h_call(x, degs_t)
    pad = jnp.arange(EPAD - E, dtype=jnp.int32) % NPAD
    srcp = jnp.concatenate([src, pad + N])
    dst2p = jnp.concatenate([dst, pad]).reshape(EPAD // BATCH, BATCH)
    hp = jnp.concatenate([h, jnp.zeros((NPAD, D), jnp.float32)], axis=0)
    aggparts = _agg_call(hp, srcp, dst2p)
    return _final_call(aggparts, degd_t, x, W, b)


# speedup vs baseline: 11.9556x; 1.3148x over previous
import dataclasses
import functools

import jax
import jax.numpy as jnp
from jax import lax
from jax.experimental import pallas as pl
from jax.experimental.pallas import tpu as pltpu
from jax.experimental.pallas import tpu_sc as plsc

N = 10000
E = 320000
D = 128

NC = 2
NS = 16
NW = NC * NS
EPW = E // NW
BATCH = 128
NFULL = EPW // BATCH
TAIL = EPW - NFULL * BATCH
RPW = 624
REM = N - NS * RPW
ZCH = 78
ECH = 2000

_mesh = lambda: plsc.VectorSubcoreMesh(core_axis_name="c", subcore_axis_name="s")


def _sc_params():
    cp = pltpu.CompilerParams()
    if "needs_layout_passes" in pltpu.CompilerParams.__dataclass_fields__:
        cp = dataclasses.replace(cp, needs_layout_passes=False)
    return cp


def _deg_call(src, dst):
    @functools.partial(
        pl.kernel,
        out_type=(
            jax.ShapeDtypeStruct((NC, NS, N), jnp.float32),
            jax.ShapeDtypeStruct((NC, NS, N), jnp.float32),
        ),
        mesh=_mesh(),
        compiler_params=_sc_params(),
        scratch_types=[
            pltpu.VMEM((ECH,), jnp.int32),
            pltpu.VMEM((ECH,), jnp.int32),
            pltpu.VMEM((N,), jnp.float32),
            pltpu.VMEM((N,), jnp.float32),
        ],
    )
    def deg_kernel(src_hbm, dst_hbm, outs_hbm, outd_hbm, sbuf, dbuf,
                   shist, dhist):
        core = lax.axis_index("c")
        sid = lax.axis_index("s")
        wid = sid * NC + core

        @pl.loop(0, N, step=16)
        def _(r):
            shist[pl.ds(r, 16)] = jnp.zeros((16,), jnp.float32)
            dhist[pl.ds(r, 16)] = jnp.zeros((16,), jnp.float32)

        ebase = wid * EPW
        ones16 = jnp.ones((16,), jnp.float32)

        @pl.loop(0, EPW // ECH)
        def _(c):
            off = ebase + c * ECH
            pltpu.sync_copy(src_hbm.at[pl.ds(off, ECH)], sbuf)
            pltpu.sync_copy(dst_hbm.at[pl.ds(off, ECH)], dbuf)

            @pl.loop(0, ECH, step=16)
            def _(k):
                plsc.addupdate_scatter(shist, [sbuf[pl.ds(k, 16)]], ones16)
                plsc.addupdate_scatter(dhist, [dbuf[pl.ds(k, 16)]], ones16)

        pltpu.sync_copy(shist, outs_hbm.at[core].at[sid])
        pltpu.sync_copy(dhist, outd_hbm.at[core].at[sid])

    return deg_kernel(src, dst)


NBW = 80
EPAD = NW * NBW * BATCH
NPAD = 8


def _agg_call(hp, srcp, dst2p):
    PH = 2
    PB = NBW // PH

    @functools.partial(
        pl.kernel,
        out_type=jax.ShapeDtypeStruct((NC, N, D), jnp.float32),
        mesh=_mesh(),
        scratch_types=[
            pltpu.VMEM((PB * BATCH,), jnp.int32),
            pltpu.VMEM((PB, BATCH), jnp.int32),
            pltpu.VMEM((BATCH, D), jnp.float32),
            pltpu.VMEM((BATCH, D), jnp.float32),
            pltpu.VMEM_SHARED((N, D), jnp.float32),
            pltpu.SemaphoreType.DMA,
            pltpu.SemaphoreType.DMA,
        ],
    )
    def agg_kernel(h_hbm, src_hbm, dst2_hbm, out_hbm, sbuf, dbuf, rows_a,
                   rows_b, agg_sh, sem_a, sem_b):
        core = lax.axis_index("c")
        sid = lax.axis_index("s")
        wid = sid * NC + core

        def gather_start(j, rows, sem):
            pltpu.async_copy(h_hbm.at[sbuf.at[pl.ds(j * BATCH, BATCH)]],
                             rows, sem)

        def gather_wait(rows, sem):
            pltpu.make_async_copy(
                h_hbm.at[sbuf.at[pl.ds(0, BATCH)]], rows, sem).wait()

        def scatter(j, rows):
            pltpu.sync_copy(rows, agg_sh.at[dbuf.at[j]], add=True)

        @pl.loop(0, BATCH)
        def _(r):
            @pl.loop(0, D, step=16)
            def _(c):
                rows_a[r, pl.ds(c, 16)] = jnp.zeros((16,), jnp.float32)

        base_row = sid * RPW

        @pl.loop(0, 4)
        def _(k):
            pltpu.sync_copy(rows_a,
                            agg_sh.at[pl.ds(base_row + k * BATCH, BATCH)])

        pltpu.sync_copy(rows_a.at[pl.ds(0, RPW - 4 * BATCH)],
                        agg_sh.at[pl.ds(base_row + 4 * BATCH,
                                        RPW - 4 * BATCH)])

        @pl.when(sid == NS - 1)
        def _():
            pltpu.sync_copy(rows_a.at[pl.ds(0, REM)],
                            agg_sh.at[pl.ds(NS * RPW, REM)])

        plsc.subcore_barrier()

        @pl.loop(0, PH)
        def _(p):
            gbase = wid * NBW + p * PB
            pltpu.sync_copy(
                src_hbm.at[pl.ds(gbase * BATCH, PB * BATCH)], sbuf)
            pltpu.sync_copy(dst2_hbm.at[pl.ds(gbase, PB)], dbuf)

            gather_start(0, rows_a, sem_a)

            @pl.loop(0, PB // 2)
            def _(i):
                j0 = 2 * i
                gather_start(j0 + 1, rows_b, sem_b)
                gather_wait(rows_a, sem_a)
                scatter(j0, rows_a)
                gather_start(jnp.minimum(j0 + 2, PB - 1), rows_a, sem_a)
                gather_wait(rows_b, sem_b)
                scatter(j0 + 1, rows_b)

            gather_wait(rows_a, sem_a)

        plsc.subcore_barrier()
        pltpu.sync_copy(agg_sh.at[pl.ds(base_row, RPW)],
                        out_hbm.at[core].at[pl.ds(base_row, RPW)])

        @pl.when(sid == NS - 1)
        def _():
            pltpu.sync_copy(agg_sh.at[pl.ds(NS * RPW, REM)],
                            out_hbm.at[core].at[pl.ds(NS * RPW, REM)])

    return agg_kernel(hp, srcp, dst2p)


_BN = 2000


def _h_body(x_ref, degp_ref, h_ref):
    deg = jnp.sum(degp_ref[...], axis=1)
    s = lax.rsqrt(jnp.maximum(deg, 1.0))
    h_ref[...] = x_ref[...] * s[:, None]


def _h_call(x, degp_t):
    return pl.pallas_call(
        _h_body,
        grid=(N // _BN,),
        in_specs=[
            pl.BlockSpec((_BN, D), lambda i: (i, 0)),
            pl.BlockSpec((_BN, NW), lambda i: (i, 0)),
        ],
        out_specs=pl.BlockSpec((_BN, D), lambda i: (i, 0)),
        out_shape=jax.ShapeDtypeStruct((N, D), jnp.float32),
    )(x, degp_t)


def _final_body(agg_ref, degp_ref, x_ref, w_ref, b_ref, o_ref):
    agg = agg_ref[0] + agg_ref[1]
    deg = jnp.sum(degp_ref[...], axis=1)
    ndst = lax.rsqrt(jnp.maximum(deg, 1.0))
    a = agg * ndst[:, None]
    x1 = jnp.dot(a, w_ref[...], preferred_element_type=jnp.float32)
    x1 = x1 + b_ref[0][None, :]
    mean = jnp.mean(x1, axis=1, keepdims=True)
    var = jnp.mean((x1 - mean) * (x1 - mean), axis=1, keepdims=True)
    x1 = (x1 - mean) * lax.rsqrt(var + 1e-5)
    o_ref[...] = jnp.maximum(x1, 0.0) + x_ref[...]


def _final_call(aggparts, degp_t, x, W, b):
    return pl.pallas_call(
        _final_body,
        grid=(N // _BN,),
        in_specs=[
            pl.BlockSpec((NC, _BN, D), lambda i: (0, i, 0)),
            pl.BlockSpec((_BN, NW), lambda i: (i, 0)),
            pl.BlockSpec((_BN, D), lambda i: (i, 0)),
            pl.BlockSpec((D, D), lambda i: (0, 0)),
            pl.BlockSpec((1, D), lambda i: (0, 0)),
        ],
        out_specs=pl.BlockSpec((_BN, D), lambda i: (i, 0)),
        out_shape=jax.ShapeDtypeStruct((N, D), jnp.float32),
    )(aggparts, degp_t, x, W, b.reshape(1, D))


def kernel(x, edge_index, W, b):
    src = edge_index[0]
    dst = edge_index[1]
    degs, degd = _deg_call(src, dst)
    degs_t = degs.reshape(NW, N).T
    degd_t = degd.reshape(NW, N).T
    h = _h_call(x, degs_t)
    pad = jnp.arange(EPAD - E, dtype=jnp.int32) % NPAD
    srcp = jnp.concatenate([src, pad + N])
    dst2p = jnp.concatenate([dst, pad]).reshape(EPAD // BATCH, BATCH)
    hp = jnp.concatenate([h, jnp.zeros((NPAD, D), jnp.float32)], axis=0)
    aggparts = _agg_call(hp, srcp, dst2p)
    return _final_call(aggparts, degd_t, x, W, b)
